# baseline (speedup 1.0000x reference)
import functools

import jax
import jax.numpy as jnp
import numpy as np
from jax import lax
from jax.experimental import pallas as pl
from jax.experimental.pallas import tpu as pltpu
from jax.experimental.pallas import tpu_sc as plsc

_VOCAB = 100000
_D = 768
_B, _S = 4, 4096
_SCALE = float(np.sqrt(np.float32(_D)))

_NC = 2
_NS = 16
_NW = _NC * _NS

_POS_PER_W = _S // _NW
_CH = 32
_NCHUNK = _POS_PER_W // _CH
_CPV = _D // 16


def _make_pe(seq_len, d):
    pos = np.arange(seq_len)[:, None].astype(np.float32)
    i = np.arange(0, d, 2).astype(np.float32)
    angle = pos / np.power(10000.0, i / np.float32(d))
    pe = np.zeros((seq_len, d), dtype=np.float32)
    pe[:, 0::2] = np.sin(angle)
    pe[:, 1::2] = np.cos(angle)
    return pe


_PE_HOST = _make_pe(_S, _D)


@functools.partial(
    pl.kernel,
    out_type=jax.ShapeDtypeStruct((_B * _S, _D), jnp.float32),
    mesh=plsc.VectorSubcoreMesh(core_axis_name="c", subcore_axis_name="s"),
    scratch_types=[
        pltpu.VMEM((_CH,), jnp.int32),
        pltpu.VMEM((_CH, _D), jnp.float32),
        pltpu.VMEM((_CH, _D), jnp.float32),
        pltpu.SemaphoreType.DMA,
    ],
)
def _emb_kernel(table_hbm, inp_hbm, pe_hbm, out_hbm, idx_v, pe_v, rows_v, sem):
    wid = lax.axis_index("s") * _NC + lax.axis_index("c")
    p_base = wid * _POS_PER_W

    for pc in range(_NCHUNK):
        pstart = p_base + pc * _CH
        pltpu.sync_copy(pe_hbm.at[pl.ds(pstart, _CH)], pe_v)
        for b in range(_B):
            rstart = b * _S + pstart
            pltpu.sync_copy(inp_hbm.at[pl.ds(rstart, _CH)], idx_v)
            pltpu.async_copy(table_hbm.at[idx_v], rows_v, sem).wait()

            def body(r, carry):
                for c in range(_CPV):
                    sl = pl.ds(c * 16, 16)
                    rows_v[r, sl] = rows_v[r, sl] * _SCALE + pe_v[r, sl]
                return carry

            lax.fori_loop(0, _CH, body, 0)
            pltpu.sync_copy(rows_v, out_hbm.at[pl.ds(rstart, _CH)])


def kernel(inp, table, is_training):
    del is_training
    pe = jnp.asarray(_PE_HOST)
    out = _emb_kernel(table, inp.reshape(_B * _S), pe)
    return out.reshape(_B, _S, _D)
